# Initial kernel scaffold; baseline (speedup 1.0000x reference)
import jax
import jax.numpy as jnp
from jax.experimental import pallas as pl


def kernel(uifs, times, edge_index, cur_len, emb_table, time_table, time_transfer, gnn1_W, gnn1_b, gnn2_W, gnn2_b, gru_Wih, gru_Whh, gru_bih, gru_bhh, fc1_W, fc1_b):
    raise NotImplementedError("write your pallas kernel here")


# speedup vs baseline: 7.9584x; 7.9584x over previous
import functools

import jax
import jax.numpy as jnp
from jax import lax
from jax.experimental import pallas as pl
from jax.experimental.pallas import tpu as pltpu
from jax.experimental.pallas import tpu_sc as plsc

NC = 2
NS = 16
NW = NC * NS
LANES = 16

ECH = 80


def _sc_gather_deg(uifs_pad, emb_table, edge_dst, n_nodes):
  npad, d = uifs_pad.shape[0], emb_table.shape[1]
  e = edge_dst.shape[0]
  rows_w = npad // NW
  edges_w = e // NW
  g_ch = rows_w // 4
  mesh = plsc.VectorSubcoreMesh(core_axis_name="c", subcore_axis_name="s")

  @functools.partial(
      pl.kernel,
      out_type=(jax.ShapeDtypeStruct((npad, d), jnp.float32),
                jax.ShapeDtypeStruct((NW, n_nodes), jnp.float32)),
      mesh=mesh,
      compiler_params=pltpu.CompilerParams(needs_layout_passes=False),
      scratch_types=[
          pltpu.VMEM((rows_w,), jnp.int32),
          pltpu.VMEM((rows_w, d), jnp.float32),
          pltpu.VMEM((edges_w,), jnp.int32),
          pltpu.VMEM((n_nodes,), jnp.float32),
          pltpu.SemaphoreType.DMA,
      ],
  )
  def k(uifs_hbm, emb_hbm, dst_hbm, x_out, deg_out, idx_v, rows_v, dst_v,
        hist_v, sem):
    c = lax.axis_index("c")
    s = lax.axis_index("s")
    w = c * NS + s
    base = w * rows_w
    pltpu.sync_copy(uifs_hbm.at[pl.ds(base, rows_w)], idx_v)
    cps = []
    for j in range(rows_w // g_ch):
      cps.append(pltpu.async_copy(
          emb_hbm.at[idx_v.at[pl.ds(j * g_ch, g_ch)]],
          rows_v.at[pl.ds(j * g_ch, g_ch)], sem))
    ebase = w * edges_w
    pltpu.sync_copy(dst_hbm.at[pl.ds(ebase, edges_w)], dst_v)
    zeros16 = jnp.zeros((LANES,), jnp.float32)
    ones16 = jnp.ones((LANES,), jnp.float32)

    def zbody(i, _):
      hist_v[pl.ds(i * LANES, LANES)] = zeros16
      return 0
    lax.fori_loop(0, n_nodes // LANES, zbody, 0)

    def hbody(i, _):
      d16 = dst_v[pl.ds(i * LANES, LANES)]
      plsc.addupdate_scatter(hist_v, [d16], ones16)
      return 0
    lax.fori_loop(0, edges_w // LANES, hbody, 0)
    pltpu.sync_copy(hist_v, deg_out.at[w])
    for cp in cps:
      cp.wait()
    pltpu.sync_copy(rows_v, x_out.at[pl.ds(base, rows_w)])

  return k(uifs_pad, emb_table, edge_dst)


def _sc_edge_accum(hs, src_r, dst_r, n_acc):
  d = hs.shape[1]
  nrows = src_r.shape[0]
  rows_w = nrows // NW
  nper = n_acc // NS
  mesh = plsc.VectorSubcoreMesh(core_axis_name="c", subcore_axis_name="s")

  @functools.partial(
      pl.kernel,
      out_type=jax.ShapeDtypeStruct((NC, n_acc, d), jnp.float32),
      mesh=mesh,
      compiler_params=pltpu.CompilerParams(needs_layout_passes=False),
      scratch_types=[
          pltpu.VMEM((rows_w, ECH), jnp.int32),
          pltpu.VMEM((rows_w, ECH), jnp.int32),
          pltpu.VMEM((ECH, d), jnp.float32),
          pltpu.VMEM_SHARED((n_acc, d), jnp.float32),
          pltpu.SemaphoreType.DMA,
      ],
  )
  def k(hs_hbm, src_hbm, dst_hbm, acc_out, src_v, dst_v, rows_v, acc_sh, sem):
    c = lax.axis_index("c")
    s = lax.axis_index("s")
    w = c * NS + s
    zeros16 = jnp.zeros((LANES,), jnp.float32)

    def zbody(i, _):
      rows_v[i // (d // LANES), pl.ds((i % (d // LANES)) * LANES, LANES)] = (
          zeros16)
      return 0
    lax.fori_loop(0, ECH * d // LANES, zbody, 0)
    for j in range(nper // ECH):
      pltpu.sync_copy(rows_v,
                      acc_sh.at[pl.ds(s * nper + j * ECH, ECH)])
    plsc.subcore_barrier()
    rbase = w * rows_w
    pltpu.sync_copy(src_hbm.at[pl.ds(rbase, rows_w)], src_v)
    pltpu.sync_copy(dst_hbm.at[pl.ds(rbase, rows_w)], dst_v)

    def ebody(j, _):
      pltpu.async_copy(hs_hbm.at[src_v.at[j]], rows_v, sem).wait()
      pltpu.sync_copy(rows_v, acc_sh.at[dst_v.at[j]], add=True)
      return 0
    lax.fori_loop(0, rows_w, ebody, 0)
    plsc.subcore_barrier()
    pltpu.sync_copy(acc_sh.at[pl.ds(s * nper, nper)],
                    acc_out.at[c, pl.ds(s * nper, nper)])

  return k(hs, src_r, dst_r)


def _tc_prep(x_raw, deg_part, times, time_table, time_transfer, w1, n_nodes):
  d = x_raw.shape[1]
  nt = times.shape[0]

  def body(x_ref, degp_ref, times_ref, tt_ref, ttr_ref, w1_ref,
           hs1_ref, dinv_ref):
    deg = jnp.sum(degp_ref[...], axis=0) + 1.0
    dinv = lax.rsqrt(deg)
    dinv_ref[...] = dinv
    rows = [tt_ref[times_ref[i], :][None, :] for i in range(nt)]
    te = jnp.concatenate(rows, axis=0) @ ttr_ref[...]
    x = jnp.concatenate([x_ref[0:n_nodes - nt], te], axis=0)
    h1 = jnp.dot(x, w1_ref[...], preferred_element_type=jnp.float32)
    hs1_ref[...] = h1 * dinv[:, None]

  return pl.pallas_call(
      body,
      out_shape=(jax.ShapeDtypeStruct((n_nodes, d), jnp.float32),
                 jax.ShapeDtypeStruct((n_nodes,), jnp.float32)),
      in_specs=[pl.BlockSpec(memory_space=pltpu.VMEM),
                pl.BlockSpec(memory_space=pltpu.VMEM),
                pl.BlockSpec(memory_space=pltpu.SMEM),
                pl.BlockSpec(memory_space=pltpu.VMEM),
                pl.BlockSpec(memory_space=pltpu.VMEM),
                pl.BlockSpec(memory_space=pltpu.VMEM)],
  )(x_raw, deg_part, times, time_table, time_transfer, w1)


def _tc_mid(acc_part, hs1, dinv, b1, w2):
  n, d = hs1.shape

  def body(accp_ref, hs1_ref, dinv_ref, b1_ref, w2_ref, hs2_ref):
    acc = accp_ref[0, 0:n] + accp_ref[1, 0:n] + hs1_ref[...]
    dinv = dinv_ref[...]
    out1 = acc * dinv[:, None] + b1_ref[...][None, :]
    h2 = jnp.dot(out1, w2_ref[...], preferred_element_type=jnp.float32)
    hs2_ref[...] = h2 * dinv[:, None]

  return pl.pallas_call(
      body,
      out_shape=jax.ShapeDtypeStruct((n, d), jnp.float32),
  )(acc_part, hs1, dinv, b1, w2)


def _tc_tail(acc2w, hs2w, dinvw, b2, wih, whh, bih, bhh, fcw, fcb):
  t, d = hs2w.shape
  h3 = wih.shape[0]

  def body(a2_ref, hs2_ref, dinv_ref, b2_ref, wih_ref, whh_ref, bih_ref,
           bhh_ref, fcw_ref, fcb_ref, out_ref, gi_ref):
    dinv = dinv_ref[...]
    seq = ((a2_ref[0] + a2_ref[1] + hs2_ref[...]) * dinv[:, None]
           + b2_ref[...][None, :])
    gi = lax.dot_general(seq, wih_ref[...], (((1,), (1,)), ((), ())),
                         preferred_element_type=jnp.float32)
    gi_ref[...] = gi + bih_ref[...][None, :]
    whh = whh_ref[...]
    bhh = bhh_ref[...][None, :]

    def step(i, h):
      git = gi_ref[pl.ds(i, 1), :]
      gh = lax.dot_general(h, whh, (((1,), (1,)), ((), ())),
                           preferred_element_type=jnp.float32) + bhh
      i_r, i_z, i_n = git[:, 0:d], git[:, d:2 * d], git[:, 2 * d:3 * d]
      h_r, h_z, h_n = gh[:, 0:d], gh[:, d:2 * d], gh[:, 2 * d:3 * d]
      r = jax.nn.sigmoid(i_r + h_r)
      z = jax.nn.sigmoid(i_z + h_z)
      nn = jnp.tanh(i_n + r * h_n)
      return (1.0 - z) * nn + z * h

    hT = lax.fori_loop(0, t, step, jnp.zeros((1, d), jnp.float32))
    out = jnp.dot(hT, fcw_ref[...].T, preferred_element_type=jnp.float32)
    out_ref[...] = jnp.maximum(out + fcb_ref[...][None, :], 0.0)

  return pl.pallas_call(
      body,
      out_shape=jax.ShapeDtypeStruct((1, d), jnp.float32),
      scratch_shapes=[pltpu.VMEM((t, h3), jnp.float32)],
  )(acc2w, hs2w, dinvw, b2, wih, whh, bih, bhh, fcw, fcb)


def kernel(uifs, times, edge_index, cur_len, emb_table, time_table,
           time_transfer, gnn1_W, gnn1_b, gnn2_W, gnn2_b, gru_Wih, gru_Whh,
           gru_bih, gru_bhh, fc1_W, fc1_b):
  n_nodes = uifs.shape[0] + times.shape[0]
  e = edge_index.shape[1]
  d = emb_table.shape[1]

  npad = ((n_nodes + NW * 8 - 1) // (NW * 8)) * (NW * 8)
  uifs_pad = jnp.concatenate(
      [uifs.astype(jnp.int32),
       jnp.zeros((npad - uifs.shape[0],), jnp.int32)])
  epad = ((e + NW * 8 * ECH - 1) // (NW * 8 * ECH)) * (NW * 8 * ECH)
  n_acc = npad
  src_pad = jnp.concatenate(
      [edge_index[0].astype(jnp.int32),
       jnp.zeros((epad - e,), jnp.int32)])
  dst_pad = jnp.concatenate(
      [edge_index[1].astype(jnp.int32),
       jnp.full((epad - e,), n_nodes, jnp.int32)])
  src_r = src_pad.reshape(epad // ECH, ECH)
  dst_r = dst_pad.reshape(epad // ECH, ECH)

  x_raw, deg_part = _sc_gather_deg(uifs_pad, emb_table,
                                   edge_index[1].astype(jnp.int32), n_nodes)
  hs1, dinv = _tc_prep(x_raw, deg_part, times.astype(jnp.int32), time_table,
                       time_transfer, gnn1_W, n_nodes)
  acc1 = _sc_edge_accum(hs1, src_r, dst_r, n_acc)
  hs2 = _tc_mid(acc1, hs1, dinv, gnn1_b, gnn2_W)
  acc2 = _sc_edge_accum(hs2, src_r, dst_r, n_acc)

  win = 51
  start = jnp.clip(jnp.asarray(cur_len, jnp.int32) - 50, 0, n_nodes - win)
  acc2w = lax.dynamic_slice(acc2, (0, start, 0), (NC, win, d))
  hs2w = lax.dynamic_slice(hs2, (start, 0), (win, d))
  dinvw = lax.dynamic_slice(dinv, (start,), (win,))
  out = _tc_tail(acc2w, hs2w, dinvw, gnn2_b, gru_Wih, gru_Whh, gru_bih,
                 gru_bhh, fc1_W, fc1_b)
  return out[:, None, :]
